# Initial kernel scaffold; baseline (speedup 1.0000x reference)
import jax
import jax.numpy as jnp
from jax.experimental import pallas as pl


def kernel(x, edge_index, batch, emb, W_conv, b_conv, W_out, b_out):
    raise NotImplementedError("write your pallas kernel here")


# speedup vs baseline: 46.6491x; 46.6491x over previous
import jax
import jax.numpy as jnp
from jax import lax
from jax.experimental import pallas as pl
from jax.experimental.pallas import tpu as pltpu
from jax.experimental.pallas import tpu_sc as plsc

N = 100000
E = 1600000
NUM_CAT = 43
NUM_GRAPHS = 64

NP = 100352
EP = 1605632
ECH = 12544
NSC = 2
NSUB = 16
NW = NSC * NSUB
NS = NP // NSUB
ACH_T = ECH // NSUB
CCH_W = ECH // NW
PADNODE = NP - 1
NBLK = NP // 512


def _deg_body(c2d, z1d, deg_out, ebuf_c, onesbuf, deg_sh, ssem):
    sub = lax.axis_index("s")
    ones16 = jnp.ones((16,), jnp.float32)
    for k in range(8):
        onesbuf[pl.ds(16 * k, 16)] = ones16
    pltpu.sync_copy(z1d, deg_sh.at[pl.ds(sub * NS, NS)])
    plsc.subcore_barrier()

    def _deg_group(g, carry):
        base = sub * ACH_T + 8 * g
        pltpu.sync_copy(c2d.at[pl.ds(base, 8)], ebuf_c)
        descs = [
            pltpu.async_copy(onesbuf, deg_sh.at[ebuf_c.at[b]], ssem, add=True)
            for b in range(8)
        ]
        for d in descs:
            d.wait()
        return carry

    lax.fori_loop(0, ACH_T // 8, _deg_group, 0)
    plsc.subcore_barrier()
    core = lax.axis_index("c")

    @pl.when(core == 0)
    def _():
        pltpu.sync_copy(deg_sh.at[pl.ds(sub * NS, NS)],
                        deg_out.at[pl.ds(sub * NS, NS)])


def _run_deg(c2d, z1d):
    mesh = plsc.VectorSubcoreMesh(core_axis_name="c", subcore_axis_name="s")
    f = pl.kernel(
        _deg_body,
        out_type=jax.ShapeDtypeStruct((NP,), jnp.float32),
        mesh=mesh,
        scratch_types=[
            pltpu.VMEM((8, 128), jnp.int32),
            pltpu.VMEM((128,), jnp.float32),
            pltpu.VMEM_SHARED((NP,), jnp.float32),
            pltpu.SemaphoreType.DMA,
        ],
        compiler_params=pltpu.CompilerParams(use_tc_tiling_on_sc=False),
    )
    return f(c2d, z1d)


def _emb_body(cat_ref, feat_ref, deg_ref, emb_ref, wc_ref, bco_ref,
              y_ref, dinv8_ref, dy_ref):
    tbl = jax.lax.dot_general(emb_ref[...], wc_ref[:32, :],
                              (((1,), (0,)), ((), ())),
                              preferred_element_type=jnp.float32)
    cat_col = cat_ref[...]
    oh = (lax.broadcasted_iota(jnp.int32, (512, 48), 1)
          == cat_col).astype(jnp.float32)
    xw = jax.lax.dot_general(oh, tbl, (((1,), (0,)), ((), ())),
                             preferred_element_type=jnp.float32)
    xw = xw + feat_ref[...] * wc_ref[32:33, :]
    dinv = jax.lax.rsqrt(deg_ref[...] + 1.0)
    y = xw * dinv
    y_ref[...] = y
    dinv8_ref[...] = jnp.broadcast_to(dinv, (512, 8))
    dy_ref[...] = dinv * y + bco_ref[0:1, :]


def _run_emb(cat2, feat2, deg2, emb48, wc33, bco8):
    spec_col = pl.BlockSpec((512, 1), lambda i: (i, 0))
    out_spec = pl.BlockSpec((512, 8), lambda i: (i, 0))
    oshape = jax.ShapeDtypeStruct((NP, 8), jnp.float32)
    return pl.pallas_call(
        _emb_body,
        grid=(NBLK,),
        in_specs=[
            spec_col,
            spec_col,
            spec_col,
            pl.BlockSpec((48, 32), lambda i: (0, 0)),
            pl.BlockSpec((33, 8), lambda i: (0, 0)),
            pl.BlockSpec((8, 8), lambda i: (0, 0)),
        ],
        out_specs=[out_spec, out_spec, out_spec],
        out_shape=[oshape, oshape, oshape],
    )(cat2, feat2, deg2, emb48, wc33, bco8)


def _msg_body(r2d, c2d, y_hbm, z2d, acc_out,
              ebuf_r, ebuf_c, gbuf, y_sh, acc_sh, gsem, ssem):
    core = lax.axis_index("c")
    sub = lax.axis_index("s")
    wid = core * NSUB + sub

    pltpu.sync_copy(y_hbm.at[pl.ds(sub * NS, NS)],
                    y_sh.at[pl.ds(sub * NS, NS)])
    for k in range(7):
        pltpu.sync_copy(z2d, acc_sh.at[pl.ds(sub * NS + 896 * k, 896)])
    plsc.subcore_barrier()

    def _msg_group(g, carry):
        base = wid * CCH_W + 8 * g
        pltpu.sync_copy(r2d.at[pl.ds(base, 8)], ebuf_r)
        pltpu.sync_copy(c2d.at[pl.ds(base, 8)], ebuf_c)
        gds = [
            pltpu.async_copy(y_sh.at[ebuf_r.at[b]],
                             gbuf.at[pl.ds(128 * b, 128)], gsem)
            for b in range(8)
        ]
        for d in gds:
            d.wait()
        sds = [
            pltpu.async_copy(gbuf.at[pl.ds(128 * b, 128)],
                             acc_sh.at[ebuf_c.at[b]], ssem, add=True)
            for b in range(8)
        ]
        for d in sds:
            d.wait()
        return carry

    lax.fori_loop(0, CCH_W // 8, _msg_group, 0)
    plsc.subcore_barrier()
    pltpu.sync_copy(acc_sh.at[pl.ds(sub * NS, NS)],
                    acc_out.at[core, pl.ds(sub * NS, NS)])


def _run_msg(r2d, c2d, y, z2d):
    mesh = plsc.VectorSubcoreMesh(core_axis_name="c", subcore_axis_name="s")
    f = pl.kernel(
        _msg_body,
        out_type=jax.ShapeDtypeStruct((NSC, NP, 8), jnp.float32),
        mesh=mesh,
        scratch_types=[
            pltpu.VMEM((8, 128), jnp.int32),
            pltpu.VMEM((8, 128), jnp.int32),
            pltpu.VMEM((1024, 8), jnp.float32),
            pltpu.VMEM_SHARED((NP, 8), jnp.float32),
            pltpu.VMEM_SHARED((NP, 8), jnp.float32),
            pltpu.SemaphoreType.DMA,
            pltpu.SemaphoreType.DMA,
        ],
        compiler_params=pltpu.CompilerParams(use_tc_tiling_on_sc=False),
    )
    return f(r2d, c2d, y, z2d)


def _pool_body(acc_ref, dinv8_ref, dy_ref, batch_ref, woutT_ref, bo_ref,
               out_ref):
    i = pl.program_id(0)
    conv = dinv8_ref[...] * (acc_ref[0] + acc_ref[1]) + dy_ref[...]
    z = jax.lax.dot_general(conv, woutT_ref[...], (((1,), (1,)), ((), ())),
                            preferred_element_type=jnp.float32)
    z = jnp.maximum(z + bo_ref[0:1, :], 0.0)
    zext = z + (lax.broadcasted_iota(jnp.int32, (512, 8), 1) == 7
                ).astype(jnp.float32)
    bt = batch_ref[0]
    oh = (lax.broadcasted_iota(jnp.int32, (NUM_GRAPHS, 512), 0)
          == bt).astype(jnp.float32)
    p = jax.lax.dot_general(oh, zext, (((1,), (0,)), ((), ())),
                            preferred_element_type=jnp.float32)

    @pl.when(i == 0)
    def _():
        out_ref[...] = p

    @pl.when(i > 0)
    def _():
        out_ref[...] = out_ref[...] + p

    @pl.when(i == NBLK - 1)
    def _():
        f = out_ref[...]
        out_ref[...] = f / jnp.maximum(f[:, 7:8], 1.0)


def _run_pool(acc2, dinv8, dy, batch3d, woutT8, bo8):
    spec8 = pl.BlockSpec((512, 8), lambda i: (i, 0))
    return pl.pallas_call(
        _pool_body,
        grid=(NBLK,),
        in_specs=[
            pl.BlockSpec((NSC, 512, 8), lambda i: (0, i, 0)),
            spec8,
            spec8,
            pl.BlockSpec((1, 1, 512), lambda i: (i, 0, 0)),
            pl.BlockSpec((8, 8), lambda i: (0, 0)),
            pl.BlockSpec((8, 8), lambda i: (0, 0)),
        ],
        out_specs=pl.BlockSpec((NUM_GRAPHS, 8), lambda i: (0, 0)),
        out_shape=jax.ShapeDtypeStruct((NUM_GRAPHS, 8), jnp.float32),
    )(acc2, dinv8, dy, batch3d, woutT8, bo8)


def kernel(x, edge_index, batch, emb, W_conv, b_conv, W_out, b_out):
    cat2 = jnp.concatenate(
        [x[:, 0].astype(jnp.int32), jnp.zeros((NP - N,), jnp.int32)]
    ).reshape(NP, 1)
    feat2 = jnp.concatenate(
        [x[:, 1], jnp.zeros((NP - N,), jnp.float32)]
    ).reshape(NP, 1)
    rows = edge_index[0].astype(jnp.int32)
    cols = edge_index[1].astype(jnp.int32)
    epad = jnp.full((EP - E,), PADNODE, jnp.int32)
    r2d = jnp.concatenate([rows, epad]).reshape(ECH, 128)
    c2d = jnp.concatenate([cols, epad]).reshape(ECH, 128)
    emb48 = jnp.zeros((48, 32), jnp.float32).at[:NUM_CAT].set(emb)
    wc33 = jnp.zeros((33, 8), jnp.float32).at[:, :7].set(W_conv)
    bco8 = jnp.zeros((8, 8), jnp.float32).at[0, :7].set(b_conv)
    z1d = jnp.zeros((NS,), jnp.float32)
    z2d = jnp.zeros((896, 8), jnp.float32)

    deg = _run_deg(c2d, z1d)
    y, dinv8, dy = _run_emb(cat2, feat2, deg.reshape(NP, 1), emb48, wc33, bco8)
    acc2 = _run_msg(r2d, c2d, y, z2d)

    batchp = jnp.concatenate(
        [batch.astype(jnp.int32), jnp.full((NP - N,), NUM_GRAPHS, jnp.int32)]
    ).reshape(NBLK, 1, 512)
    woutT8 = jnp.zeros((8, 8), jnp.float32).at[:7, :7].set(W_out.T)
    bo8 = jnp.zeros((8, 8), jnp.float32).at[0, :7].set(b_out)

    pooled = _run_pool(acc2, dinv8, dy, batchp, woutT8, bo8)
    return pooled[:, :7]


# speedup vs baseline: 57.4268x; 1.2310x over previous
import jax
import jax.numpy as jnp
from jax import lax
from jax.experimental import pallas as pl
from jax.experimental.pallas import tpu as pltpu
from jax.experimental.pallas import tpu_sc as plsc

N = 100000
E = 1600000
NUM_CAT = 43
NUM_GRAPHS = 64

NP = 100352
ECH = E // 128
NSC = 2
NSUB = 16
NW = NSC * NSUB
NS = NP // NSUB
BLK = 2048
NBLK = NP // BLK


def _deg_body(c2d, ones_in, z1d, deg_out, ebuf_c, onesbuf, deg_sh, ssem):
    sub = lax.axis_index("s")
    pltpu.sync_copy(ones_in, onesbuf)
    pltpu.sync_copy(z1d, deg_sh.at[pl.ds(sub * NS, NS)])
    plsc.subcore_barrier()

    c0 = (ECH * sub) // NSUB
    c1 = (ECH * (sub + 1)) // NSUB
    n8 = (c1 - c0) // 8

    def _deg_group(g, carry):
        base = c0 + 8 * g
        pltpu.sync_copy(c2d.at[pl.ds(base, 8)], ebuf_c)
        descs = [
            pltpu.async_copy(onesbuf, deg_sh.at[ebuf_c.at[b]], ssem, add=True)
            for b in range(8)
        ]
        for d in descs:
            d.wait()
        return carry

    lax.fori_loop(0, n8, _deg_group, 0)

    def _deg_tail(i, carry):
        pltpu.sync_copy(c2d.at[pl.ds(c0 + 8 * n8 + i, 1)],
                        ebuf_c.at[pl.ds(0, 1)])
        pltpu.async_copy(onesbuf, deg_sh.at[ebuf_c.at[0]], ssem,
                         add=True).wait()
        return carry

    lax.fori_loop(0, (c1 - c0) - 8 * n8, _deg_tail, 0)
    plsc.subcore_barrier()
    core = lax.axis_index("c")

    @pl.when(core == 0)
    def _():
        pltpu.sync_copy(deg_sh.at[pl.ds(sub * NS, NS)],
                        deg_out.at[pl.ds(sub * NS, NS)])


def _run_deg(c2d, ones_in, z1d):
    mesh = plsc.VectorSubcoreMesh(core_axis_name="c", subcore_axis_name="s")
    f = pl.kernel(
        _deg_body,
        out_type=jax.ShapeDtypeStruct((NP, 1), jnp.float32),
        mesh=mesh,
        scratch_types=[
            pltpu.VMEM((8, 128), jnp.int32),
            pltpu.VMEM((128, 1), jnp.float32),
            pltpu.VMEM_SHARED((NP, 1), jnp.float32),
            pltpu.SemaphoreType.DMA,
        ],
        compiler_params=pltpu.CompilerParams(use_tc_tiling_on_sc=False),
    )
    return f(c2d, ones_in, z1d)


def _emb_body(cat_ref, feat_ref, deg_ref, emb_ref, wc_ref, bco_ref,
              y_ref, dinv8_ref, dy_ref):
    tbl = jax.lax.dot_general(emb_ref[...], wc_ref[:32, :],
                              (((1,), (0,)), ((), ())),
                              preferred_element_type=jnp.float32)
    cat_col = cat_ref[...]
    oh = (lax.broadcasted_iota(jnp.int32, (BLK, 48), 1)
          == cat_col).astype(jnp.float32)
    xw = jax.lax.dot_general(oh, tbl, (((1,), (0,)), ((), ())),
                             preferred_element_type=jnp.float32)
    xw = xw + feat_ref[...] * wc_ref[32:33, :]
    dinv = jax.lax.rsqrt(deg_ref[...] + 1.0)
    y = xw * dinv
    y_ref[...] = y
    dinv8_ref[...] = jnp.broadcast_to(dinv, (BLK, 8))
    dy_ref[...] = dinv * y + bco_ref[0:1, :]


def _run_emb(cat2, feat2, deg2, emb48, wc33, bco8):
    spec_col = pl.BlockSpec((BLK, 1), lambda i: (i, 0))
    out_spec = pl.BlockSpec((BLK, 8), lambda i: (i, 0))
    oshape = jax.ShapeDtypeStruct((NP, 8), jnp.float32)
    return pl.pallas_call(
        _emb_body,
        grid=(NBLK,),
        in_specs=[
            spec_col,
            spec_col,
            spec_col,
            pl.BlockSpec((48, 32), lambda i: (0, 0)),
            pl.BlockSpec((33, 8), lambda i: (0, 0)),
            pl.BlockSpec((8, 8), lambda i: (0, 0)),
        ],
        out_specs=[out_spec, out_spec, out_spec],
        out_shape=[oshape, oshape, oshape],
    )(cat2, feat2, deg2, emb48, wc33, bco8)


def _msg_body(r2d, c2d, y_hbm, z2d, acc_out,
              ebuf_r, ebuf_c, gbuf, y_sh, acc_sh, gsem, ssem):
    core = lax.axis_index("c")
    sub = lax.axis_index("s")
    wid = core * NSUB + sub

    pltpu.sync_copy(y_hbm.at[pl.ds(sub * NS, NS)],
                    y_sh.at[pl.ds(sub * NS, NS)])
    for k in range(7):
        pltpu.sync_copy(z2d, acc_sh.at[pl.ds(sub * NS + 896 * k, 896)])
    plsc.subcore_barrier()

    c0 = (ECH * wid) // NW
    c1 = (ECH * (wid + 1)) // NW
    n8 = (c1 - c0) // 8

    def _msg_group(g, carry):
        base = c0 + 8 * g
        pltpu.sync_copy(r2d.at[pl.ds(base, 8)], ebuf_r)
        pltpu.sync_copy(c2d.at[pl.ds(base, 8)], ebuf_c)
        gds = [
            pltpu.async_copy(y_sh.at[ebuf_r.at[b]],
                             gbuf.at[pl.ds(128 * b, 128)], gsem)
            for b in range(8)
        ]
        for d in gds:
            d.wait()
        sds = [
            pltpu.async_copy(gbuf.at[pl.ds(128 * b, 128)],
                             acc_sh.at[ebuf_c.at[b]], ssem, add=True)
            for b in range(8)
        ]
        for d in sds:
            d.wait()
        return carry

    lax.fori_loop(0, n8, _msg_group, 0)

    def _msg_tail(i, carry):
        pltpu.sync_copy(r2d.at[pl.ds(c0 + 8 * n8 + i, 1)],
                        ebuf_r.at[pl.ds(0, 1)])
        pltpu.sync_copy(c2d.at[pl.ds(c0 + 8 * n8 + i, 1)],
                        ebuf_c.at[pl.ds(0, 1)])
        pltpu.async_copy(y_sh.at[ebuf_r.at[0]], gbuf.at[pl.ds(0, 128)],
                         gsem).wait()
        pltpu.async_copy(gbuf.at[pl.ds(0, 128)], acc_sh.at[ebuf_c.at[0]],
                         ssem, add=True).wait()
        return carry

    lax.fori_loop(0, (c1 - c0) - 8 * n8, _msg_tail, 0)
    plsc.subcore_barrier()
    pltpu.sync_copy(acc_sh.at[pl.ds(sub * NS, NS)],
                    acc_out.at[core, pl.ds(sub * NS, NS)])


def _run_msg(r2d, c2d, y, z2d):
    mesh = plsc.VectorSubcoreMesh(core_axis_name="c", subcore_axis_name="s")
    f = pl.kernel(
        _msg_body,
        out_type=jax.ShapeDtypeStruct((NSC, NP, 8), jnp.float32),
        mesh=mesh,
        scratch_types=[
            pltpu.VMEM((8, 128), jnp.int32),
            pltpu.VMEM((8, 128), jnp.int32),
            pltpu.VMEM((1024, 8), jnp.float32),
            pltpu.VMEM_SHARED((NP, 8), jnp.float32),
            pltpu.VMEM_SHARED((NP, 8), jnp.float32),
            pltpu.SemaphoreType.DMA,
            pltpu.SemaphoreType.DMA,
        ],
        compiler_params=pltpu.CompilerParams(use_tc_tiling_on_sc=False),
    )
    return f(r2d, c2d, y, z2d)


def _pool_body(acc_ref, dinv8_ref, dy_ref, batch_ref, woutT_ref, bo_ref,
               out_ref):
    i = pl.program_id(0)
    conv = dinv8_ref[...] * (acc_ref[0] + acc_ref[1]) + dy_ref[...]
    z = jax.lax.dot_general(conv, woutT_ref[...], (((1,), (1,)), ((), ())),
                            preferred_element_type=jnp.float32)
    z = jnp.maximum(z + bo_ref[0:1, :], 0.0)
    zext = z + (lax.broadcasted_iota(jnp.int32, (BLK, 8), 1) == 7
                ).astype(jnp.float32)
    bt = batch_ref[0]
    oh = (lax.broadcasted_iota(jnp.int32, (NUM_GRAPHS, BLK), 0)
          == bt).astype(jnp.float32)
    p = jax.lax.dot_general(oh, zext, (((1,), (0,)), ((), ())),
                            preferred_element_type=jnp.float32)

    @pl.when(i == 0)
    def _():
        out_ref[...] = p

    @pl.when(i > 0)
    def _():
        out_ref[...] = out_ref[...] + p

    @pl.when(i == NBLK - 1)
    def _():
        f = out_ref[...]
        out_ref[...] = f / jnp.maximum(f[:, 7:8], 1.0)


def _run_pool(acc2, dinv8, dy, batch3d, woutT8, bo8):
    spec8 = pl.BlockSpec((BLK, 8), lambda i: (i, 0))
    return pl.pallas_call(
        _pool_body,
        grid=(NBLK,),
        in_specs=[
            pl.BlockSpec((NSC, BLK, 8), lambda i: (0, i, 0)),
            spec8,
            spec8,
            pl.BlockSpec((1, 1, BLK), lambda i: (i, 0, 0)),
            pl.BlockSpec((8, 8), lambda i: (0, 0)),
            pl.BlockSpec((8, 8), lambda i: (0, 0)),
        ],
        out_specs=pl.BlockSpec((NUM_GRAPHS, 8), lambda i: (0, 0)),
        out_shape=jax.ShapeDtypeStruct((NUM_GRAPHS, 8), jnp.float32),
    )(acc2, dinv8, dy, batch3d, woutT8, bo8)


def kernel(x, edge_index, batch, emb, W_conv, b_conv, W_out, b_out):
    cat2 = jnp.concatenate(
        [x[:, 0].astype(jnp.int32), jnp.zeros((NP - N,), jnp.int32)]
    ).reshape(NP, 1)
    feat2 = jnp.concatenate(
        [x[:, 1], jnp.zeros((NP - N,), jnp.float32)]
    ).reshape(NP, 1)
    r2d = edge_index[0].astype(jnp.int32).reshape(ECH, 128)
    c2d = edge_index[1].astype(jnp.int32).reshape(ECH, 128)
    emb48 = jnp.zeros((48, 32), jnp.float32).at[:NUM_CAT].set(emb)
    wc33 = jnp.zeros((33, 8), jnp.float32).at[:, :7].set(W_conv)
    bco8 = jnp.zeros((8, 8), jnp.float32).at[0, :7].set(b_conv)
    ones_in = jnp.ones((128, 1), jnp.float32)
    z1d = jnp.zeros((NS, 1), jnp.float32)
    z2d = jnp.zeros((896, 8), jnp.float32)

    deg2 = _run_deg(c2d, ones_in, z1d)
    y, dinv8, dy = _run_emb(cat2, feat2, deg2, emb48, wc33, bco8)
    acc2 = _run_msg(r2d, c2d, y, z2d)

    batchp = jnp.concatenate(
        [batch.astype(jnp.int32), jnp.full((NP - N,), NUM_GRAPHS, jnp.int32)]
    ).reshape(NBLK, 1, BLK)
    woutT8 = jnp.zeros((8, 8), jnp.float32).at[:7, :7].set(W_out.T)
    bo8 = jnp.zeros((8, 8), jnp.float32).at[0, :7].set(b_out)

    pooled = _run_pool(acc2, dinv8, dy, batchp, woutT8, bo8)
    return pooled[:, :7]


# speedup vs baseline: 61.8869x; 1.0777x over previous
import jax
import jax.numpy as jnp
from jax import lax
from jax.experimental import pallas as pl
from jax.experimental.pallas import tpu as pltpu
from jax.experimental.pallas import tpu_sc as plsc

N = 100000
E = 1600000
NUM_CAT = 43
NUM_GRAPHS = 64

NP = 100352
ECH = E // 128
NSC = 2
NSUB = 16
NW = NSC * NSUB
NS = NP // NSUB
BLK = 2048
NBLK = NP // BLK


def _deg_body(ei3, ones_in, z2d, deg_out, ebuf_c, onesbuf, deg_sh, ssem):
    sub = lax.axis_index("s")
    pltpu.sync_copy(ones_in, onesbuf)
    for k in range(7):
        pltpu.sync_copy(z2d, deg_sh.at[pl.ds(sub * NS + 896 * k, 896)])
    plsc.subcore_barrier()

    c0 = (ECH * sub) // NSUB
    c1 = (ECH * (sub + 1)) // NSUB
    n8 = (c1 - c0) // 8

    def _deg_group(g, carry):
        base = c0 + 8 * g
        pltpu.sync_copy(ei3.at[1, pl.ds(base, 8)], ebuf_c)
        descs = [
            pltpu.async_copy(onesbuf, deg_sh.at[ebuf_c.at[b]], ssem, add=True)
            for b in range(8)
        ]
        for d in descs:
            d.wait()
        return carry

    lax.fori_loop(0, n8, _deg_group, 0)

    def _deg_tail(i, carry):
        pltpu.sync_copy(ei3.at[1, pl.ds(c0 + 8 * n8 + i, 1)],
                        ebuf_c.at[pl.ds(0, 1)])
        pltpu.async_copy(onesbuf, deg_sh.at[ebuf_c.at[0]], ssem,
                         add=True).wait()
        return carry

    lax.fori_loop(0, (c1 - c0) - 8 * n8, _deg_tail, 0)
    plsc.subcore_barrier()
    core = lax.axis_index("c")

    @pl.when(core == 0)
    def _():
        pltpu.sync_copy(deg_sh.at[pl.ds(sub * NS, NS)],
                        deg_out.at[pl.ds(sub * NS, NS)])


def _run_deg(ei3, ones_in, z2d):
    mesh = plsc.VectorSubcoreMesh(core_axis_name="c", subcore_axis_name="s")
    f = pl.kernel(
        _deg_body,
        out_type=jax.ShapeDtypeStruct((NP, 8), jnp.float32),
        mesh=mesh,
        scratch_types=[
            pltpu.VMEM((8, 128), jnp.int32),
            pltpu.VMEM((128, 8), jnp.float32),
            pltpu.VMEM_SHARED((NP, 8), jnp.float32),
            pltpu.SemaphoreType.DMA,
        ],
        compiler_params=pltpu.CompilerParams(use_tc_tiling_on_sc=False),
    )
    return f(ei3, ones_in, z2d)


def _emb_body(cat_ref, feat_ref, deg8_ref, emb_ref, wc_ref, y_ref):
    tbl = jax.lax.dot_general(emb_ref[...], wc_ref[:32, :],
                              (((1,), (0,)), ((), ())),
                              preferred_element_type=jnp.float32)
    cat_col = cat_ref[...]
    oh = (lax.broadcasted_iota(jnp.int32, (BLK, 48), 1)
          == cat_col).astype(jnp.float32)
    xw = jax.lax.dot_general(oh, tbl, (((1,), (0,)), ((), ())),
                             preferred_element_type=jnp.float32)
    xw = xw + feat_ref[...] * wc_ref[32:33, :]
    y_ref[...] = xw * jax.lax.rsqrt(deg8_ref[...] + 1.0)


def _run_emb(cat2, feat2, deg8, emb48, wc33):
    spec_col = pl.BlockSpec((BLK, 1), lambda i: (i, 0))
    return pl.pallas_call(
        _emb_body,
        grid=(NBLK,),
        in_specs=[
            spec_col,
            spec_col,
            pl.BlockSpec((BLK, 8), lambda i: (i, 0)),
            pl.BlockSpec((48, 32), lambda i: (0, 0)),
            pl.BlockSpec((33, 8), lambda i: (0, 0)),
        ],
        out_specs=pl.BlockSpec((BLK, 8), lambda i: (i, 0)),
        out_shape=jax.ShapeDtypeStruct((NP, 8), jnp.float32),
    )(cat2, feat2, deg8, emb48, wc33)


def _msg_body(ei3, y_hbm, z2d, acc_out,
              ebuf_r, ebuf_c, gbuf, y_sh, acc_sh, gsem, ssem):
    core = lax.axis_index("c")
    sub = lax.axis_index("s")
    wid = core * NSUB + sub

    pltpu.sync_copy(y_hbm.at[pl.ds(sub * NS, NS)],
                    y_sh.at[pl.ds(sub * NS, NS)])
    for k in range(7):
        pltpu.sync_copy(z2d, acc_sh.at[pl.ds(sub * NS + 896 * k, 896)])
    plsc.subcore_barrier()

    c0 = (ECH * wid) // NW
    c1 = (ECH * (wid + 1)) // NW
    n8 = (c1 - c0) // 8

    def _msg_group(g, carry):
        base = c0 + 8 * g
        pltpu.sync_copy(ei3.at[0, pl.ds(base, 8)], ebuf_r)
        pltpu.sync_copy(ei3.at[1, pl.ds(base, 8)], ebuf_c)
        gds = [
            pltpu.async_copy(y_sh.at[ebuf_r.at[b]],
                             gbuf.at[pl.ds(128 * b, 128)], gsem)
            for b in range(8)
        ]
        for d in gds:
            d.wait()
        sds = [
            pltpu.async_copy(gbuf.at[pl.ds(128 * b, 128)],
                             acc_sh.at[ebuf_c.at[b]], ssem, add=True)
            for b in range(8)
        ]
        for d in sds:
            d.wait()
        return carry

    lax.fori_loop(0, n8, _msg_group, 0)

    def _msg_tail(i, carry):
        pltpu.sync_copy(ei3.at[0, pl.ds(c0 + 8 * n8 + i, 1)],
                        ebuf_r.at[pl.ds(0, 1)])
        pltpu.sync_copy(ei3.at[1, pl.ds(c0 + 8 * n8 + i, 1)],
                        ebuf_c.at[pl.ds(0, 1)])
        pltpu.async_copy(y_sh.at[ebuf_r.at[0]], gbuf.at[pl.ds(0, 128)],
                         gsem).wait()
        pltpu.async_copy(gbuf.at[pl.ds(0, 128)], acc_sh.at[ebuf_c.at[0]],
                         ssem, add=True).wait()
        return carry

    lax.fori_loop(0, (c1 - c0) - 8 * n8, _msg_tail, 0)
    plsc.subcore_barrier()
    pltpu.sync_copy(acc_sh.at[pl.ds(sub * NS, NS)],
                    acc_out.at[core, pl.ds(sub * NS, NS)])


def _run_msg(ei3, y, z2d):
    mesh = plsc.VectorSubcoreMesh(core_axis_name="c", subcore_axis_name="s")
    f = pl.kernel(
        _msg_body,
        out_type=jax.ShapeDtypeStruct((NSC, NP, 8), jnp.float32),
        mesh=mesh,
        scratch_types=[
            pltpu.VMEM((8, 128), jnp.int32),
            pltpu.VMEM((8, 128), jnp.int32),
            pltpu.VMEM((1024, 8), jnp.float32),
            pltpu.VMEM_SHARED((NP, 8), jnp.float32),
            pltpu.VMEM_SHARED((NP, 8), jnp.float32),
            pltpu.SemaphoreType.DMA,
            pltpu.SemaphoreType.DMA,
        ],
        compiler_params=pltpu.CompilerParams(use_tc_tiling_on_sc=False),
    )
    return f(ei3, y, z2d)


def _pool_body(acc_ref, deg8_ref, y_ref, batch_ref, woutT_ref, bco_ref,
               bo_ref, out_ref):
    i = pl.program_id(0)
    dinv8 = jax.lax.rsqrt(deg8_ref[...] + 1.0)
    conv = dinv8 * (acc_ref[0] + acc_ref[1] + y_ref[...]) + bco_ref[0:1, :]
    z = jax.lax.dot_general(conv, woutT_ref[...], (((1,), (1,)), ((), ())),
                            preferred_element_type=jnp.float32)
    z = jnp.maximum(z + bo_ref[0:1, :], 0.0)
    zext = z + (lax.broadcasted_iota(jnp.int32, (BLK, 8), 1) == 7
                ).astype(jnp.float32)
    bt = batch_ref[0]
    oh = (lax.broadcasted_iota(jnp.int32, (NUM_GRAPHS, BLK), 0)
          == bt).astype(jnp.float32)
    p = jax.lax.dot_general(oh, zext, (((1,), (0,)), ((), ())),
                            preferred_element_type=jnp.float32)

    @pl.when(i == 0)
    def _():
        out_ref[...] = p

    @pl.when(i > 0)
    def _():
        out_ref[...] = out_ref[...] + p

    @pl.when(i == NBLK - 1)
    def _():
        f = out_ref[...]
        out_ref[...] = f / jnp.maximum(f[:, 7:8], 1.0)


def _run_pool(acc2, deg8, y, batch3d, woutT8, bco8, bo8):
    spec8 = pl.BlockSpec((BLK, 8), lambda i: (i, 0))
    return pl.pallas_call(
        _pool_body,
        grid=(NBLK,),
        in_specs=[
            pl.BlockSpec((NSC, BLK, 8), lambda i: (0, i, 0)),
            spec8,
            spec8,
            pl.BlockSpec((1, 1, BLK), lambda i: (i, 0, 0)),
            pl.BlockSpec((8, 8), lambda i: (0, 0)),
            pl.BlockSpec((8, 8), lambda i: (0, 0)),
            pl.BlockSpec((8, 8), lambda i: (0, 0)),
        ],
        out_specs=pl.BlockSpec((NUM_GRAPHS, 8), lambda i: (0, 0)),
        out_shape=jax.ShapeDtypeStruct((NUM_GRAPHS, 8), jnp.float32),
    )(acc2, deg8, y, batch3d, woutT8, bco8, bo8)


def kernel(x, edge_index, batch, emb, W_conv, b_conv, W_out, b_out):
    cat2 = jnp.concatenate(
        [x[:, 0].astype(jnp.int32), jnp.zeros((NP - N,), jnp.int32)]
    ).reshape(NP, 1)
    feat2 = jnp.concatenate(
        [x[:, 1], jnp.zeros((NP - N,), jnp.float32)]
    ).reshape(NP, 1)
    ei3 = edge_index.astype(jnp.int32).reshape(2, ECH, 128)
    emb48 = jnp.zeros((48, 32), jnp.float32).at[:NUM_CAT].set(emb)
    wc33 = jnp.zeros((33, 8), jnp.float32).at[:, :7].set(W_conv)
    ones_in = jnp.ones((128, 8), jnp.float32)
    z2d = jnp.zeros((896, 8), jnp.float32)

    deg8 = _run_deg(ei3, ones_in, z2d)
    y = _run_emb(cat2, feat2, deg8, emb48, wc33)
    acc2 = _run_msg(ei3, y, z2d)

    batchp = jnp.concatenate(
        [batch.astype(jnp.int32), jnp.full((NP - N,), NUM_GRAPHS, jnp.int32)]
    ).reshape(NBLK, 1, BLK)
    woutT8 = jnp.zeros((8, 8), jnp.float32).at[:7, :7].set(W_out.T)
    bco8 = jnp.zeros((8, 8), jnp.float32).at[0, :7].set(b_conv)
    bo8 = jnp.zeros((8, 8), jnp.float32).at[0, :7].set(b_out)

    pooled = _run_pool(acc2, deg8, y, batchp, woutT8, bco8, bo8)
    return pooled[:, :7]


# speedup vs baseline: 98.2312x; 1.5873x over previous
import jax
import jax.numpy as jnp
from jax import lax
from jax.experimental import pallas as pl
from jax.experimental.pallas import tpu as pltpu
from jax.experimental.pallas import tpu_sc as plsc

N = 100000
E = 1600000
NUM_CAT = 43
NUM_GRAPHS = 64

NP = 100352
ECH = E // 128
NSC = 2
NSUB = 16
NW = NSC * NSUB
NS = NP // NSUB
RT = NP * 8 // 128
RB = 128
NBLK = RT // RB


def _deg_body(ei3, ones_in, z1d, deg_out, ebuf_c, onesbuf, deg_sh, ssem):
    sub = lax.axis_index("s")
    pltpu.sync_copy(ones_in, onesbuf)
    pltpu.sync_copy(z1d, deg_sh.at[pl.ds(sub * NS, NS)])
    plsc.subcore_barrier()

    c0 = (ECH * sub) // NSUB
    c1 = (ECH * (sub + 1)) // NSUB
    n8 = (c1 - c0) // 8

    def _deg_group(g, carry):
        base = c0 + 8 * g
        pltpu.sync_copy(ei3.at[1, pl.ds(base, 8)], ebuf_c)
        descs = [
            pltpu.async_copy(onesbuf, deg_sh.at[ebuf_c.at[b]], ssem, add=True)
            for b in range(8)
        ]
        for d in descs:
            d.wait()
        return carry

    lax.fori_loop(0, n8, _deg_group, 0)

    def _deg_tail(i, carry):
        pltpu.sync_copy(ei3.at[1, pl.ds(c0 + 8 * n8 + i, 1)],
                        ebuf_c.at[pl.ds(0, 1)])
        pltpu.async_copy(onesbuf, deg_sh.at[ebuf_c.at[0]], ssem,
                         add=True).wait()
        return carry

    lax.fori_loop(0, (c1 - c0) - 8 * n8, _deg_tail, 0)
    plsc.subcore_barrier()
    core = lax.axis_index("c")

    @pl.when(core == 0)
    def _():
        pltpu.sync_copy(deg_sh.at[pl.ds(sub * NS, NS)],
                        deg_out.at[pl.ds(sub * NS, NS)])


def _run_deg(ei3, ones_in, z1d):
    mesh = plsc.VectorSubcoreMesh(core_axis_name="c", subcore_axis_name="s")
    f = pl.kernel(
        _deg_body,
        out_type=jax.ShapeDtypeStruct((NP,), jnp.float32),
        mesh=mesh,
        scratch_types=[
            pltpu.VMEM((8, 128), jnp.int32),
            pltpu.VMEM((128,), jnp.float32),
            pltpu.VMEM_SHARED((NP,), jnp.float32),
            pltpu.SemaphoreType.DMA,
        ],
        compiler_params=pltpu.CompilerParams(use_tc_tiling_on_sc=False),
    )
    return f(ei3, ones_in, z1d)


def _emb_body(cat_ref, feat_ref, deg_ref, emb_ref, wc_ref, wf_ref,
              p_ref, u_ref, v_ref, dmask_ref, y_ref):
    tbl = jax.lax.dot_general(emb_ref[...], wc_ref[:32, :],
                              (((1,), (0,)), ((), ())),
                              preferred_element_type=jnp.float32)
    tiled = jax.lax.dot_general(
        jax.lax.dot_general(u_ref[...], tbl, (((1,), (0,)), ((), ())),
                            preferred_element_type=jnp.float32),
        v_ref[...], (((1,), (0,)), ((), ())),
        preferred_element_type=jnp.float32)
    tblbd = tiled * dmask_ref[...]
    catf = cat_ref[...].astype(jnp.float32)
    catrep = jax.lax.dot_general(catf, p_ref[...], (((1,), (0,)), ((), ())),
                                 preferred_element_type=jnp.float32)
    mod48 = (lax.broadcasted_iota(jnp.int32, (RB, 768), 1) % 48
             ).astype(jnp.float32)
    ohbig = (catrep == mod48).astype(jnp.float32)
    y0 = jax.lax.dot_general(ohbig, tblbd, (((1,), (0,)), ((), ())),
                             preferred_element_type=jnp.float32)
    dinv = jax.lax.rsqrt(deg_ref[...] + 1.0)
    y_ref[...] = (y0 + feat_ref[...] * wf_ref[...]) * dinv


def _run_emb(cat128, feat128, deg128, emb48, wc33, wf128, pmat, umat, vmat,
             dmask):
    spec = pl.BlockSpec((RB, 128), lambda i: (i, 0))
    return pl.pallas_call(
        _emb_body,
        grid=(NBLK,),
        in_specs=[
            spec,
            spec,
            spec,
            pl.BlockSpec((48, 32), lambda i: (0, 0)),
            pl.BlockSpec((33, 8), lambda i: (0, 0)),
            pl.BlockSpec((1, 128), lambda i: (0, 0)),
            pl.BlockSpec((128, 768), lambda i: (0, 0)),
            pl.BlockSpec((768, 48), lambda i: (0, 0)),
            pl.BlockSpec((8, 128), lambda i: (0, 0)),
            pl.BlockSpec((768, 128), lambda i: (0, 0)),
        ],
        out_specs=spec,
        out_shape=jax.ShapeDtypeStruct((RT, 128), jnp.float32),
    )(cat128, feat128, deg128, emb48, wc33, wf128, pmat, umat, vmat, dmask)


def _msg_body(ei3, y_hbm, z2d, acc_out,
              ebuf_r, ebuf_c, gbuf, y_sh, acc_sh, gsem, ssem):
    core = lax.axis_index("c")
    sub = lax.axis_index("s")
    wid = core * NSUB + sub

    pltpu.sync_copy(y_hbm.at[pl.ds(sub * NS, NS)],
                    y_sh.at[pl.ds(sub * NS, NS)])
    for k in range(7):
        pltpu.sync_copy(z2d, acc_sh.at[pl.ds(sub * NS + 896 * k, 896)])
    plsc.subcore_barrier()

    c0 = (ECH * wid) // NW
    c1 = (ECH * (wid + 1)) // NW
    n8 = (c1 - c0) // 8

    def _msg_group(g, carry):
        base = c0 + 8 * g
        pltpu.sync_copy(ei3.at[0, pl.ds(base, 8)], ebuf_r)
        pltpu.sync_copy(ei3.at[1, pl.ds(base, 8)], ebuf_c)
        gds = [
            pltpu.async_copy(y_sh.at[ebuf_r.at[b]],
                             gbuf.at[pl.ds(128 * b, 128)], gsem)
            for b in range(8)
        ]
        for d in gds:
            d.wait()
        sds = [
            pltpu.async_copy(gbuf.at[pl.ds(128 * b, 128)],
                             acc_sh.at[ebuf_c.at[b]], ssem, add=True)
            for b in range(8)
        ]
        for d in sds:
            d.wait()
        return carry

    lax.fori_loop(0, n8, _msg_group, 0)

    def _msg_tail(i, carry):
        pltpu.sync_copy(ei3.at[0, pl.ds(c0 + 8 * n8 + i, 1)],
                        ebuf_r.at[pl.ds(0, 1)])
        pltpu.sync_copy(ei3.at[1, pl.ds(c0 + 8 * n8 + i, 1)],
                        ebuf_c.at[pl.ds(0, 1)])
        pltpu.async_copy(y_sh.at[ebuf_r.at[0]], gbuf.at[pl.ds(0, 128)],
                         gsem).wait()
        pltpu.async_copy(gbuf.at[pl.ds(0, 128)], acc_sh.at[ebuf_c.at[0]],
                         ssem, add=True).wait()
        return carry

    lax.fori_loop(0, (c1 - c0) - 8 * n8, _msg_tail, 0)
    plsc.subcore_barrier()
    pltpu.sync_copy(acc_sh.at[pl.ds(sub * NS, NS)],
                    acc_out.at[core, pl.ds(sub * NS, NS)])


def _run_msg(ei3, y, z2d):
    mesh = plsc.VectorSubcoreMesh(core_axis_name="c", subcore_axis_name="s")
    f = pl.kernel(
        _msg_body,
        out_type=jax.ShapeDtypeStruct((NSC, NP, 8), jnp.float32),
        mesh=mesh,
        scratch_types=[
            pltpu.VMEM((8, 128), jnp.int32),
            pltpu.VMEM((8, 128), jnp.int32),
            pltpu.VMEM((1024, 8), jnp.float32),
            pltpu.VMEM_SHARED((NP, 8), jnp.float32),
            pltpu.VMEM_SHARED((NP, 8), jnp.float32),
            pltpu.SemaphoreType.DMA,
            pltpu.SemaphoreType.DMA,
        ],
        compiler_params=pltpu.CompilerParams(use_tc_tiling_on_sc=False),
    )
    return f(ei3, y, z2d)


def _pool_body(acc_ref, y_ref, deg_ref, batch_ref, wout_ref, u8_ref, v_ref,
               dmask8_ref, bco_ref, bo_ref, out_ref):
    i = pl.program_id(0)
    dinv = jax.lax.rsqrt(deg_ref[...] + 1.0)
    conv = dinv * (acc_ref[0] + acc_ref[1] + y_ref[...]) + bco_ref[...]
    wbd = jax.lax.dot_general(
        jax.lax.dot_general(u8_ref[...], wout_ref[...],
                            (((1,), (0,)), ((), ())),
                            preferred_element_type=jnp.float32),
        v_ref[...], (((1,), (0,)), ((), ())),
        preferred_element_type=jnp.float32) * dmask8_ref[...]
    z = jax.lax.dot_general(conv, wbd, (((1,), (0,)), ((), ())),
                            preferred_element_type=jnp.float32)
    z = jnp.maximum(z + bo_ref[...], 0.0)
    zext = z + (lax.broadcasted_iota(jnp.int32, (RB, 128), 1) % 8 == 7
                ).astype(jnp.float32)
    iota64 = lax.broadcasted_iota(jnp.int32, (1, NUM_GRAPHS), 1)
    p = jnp.zeros((NUM_GRAPHS, 8), jnp.float32)
    for s in range(16):
        z_s = zext[:, 8 * s:8 * s + 8]
        b_s = batch_ref[:, 8 * s:8 * s + 1]
        oh_s = (b_s == iota64).astype(jnp.float32)
        p = p + jax.lax.dot_general(oh_s, z_s, (((0,), (0,)), ((), ())),
                                    preferred_element_type=jnp.float32)

    @pl.when(i == 0)
    def _():
        out_ref[...] = p

    @pl.when(i > 0)
    def _():
        out_ref[...] = out_ref[...] + p

    @pl.when(i == NBLK - 1)
    def _():
        f = out_ref[...]
        out_ref[...] = f / jnp.maximum(f[:, 7:8], 1.0)


def _run_pool(acc128, y128, deg128, batch128, wout8, u8, v8, dmask8, bco128,
              bo128):
    spec = pl.BlockSpec((RB, 128), lambda i: (i, 0))
    return pl.pallas_call(
        _pool_body,
        grid=(NBLK,),
        in_specs=[
            pl.BlockSpec((NSC, RB, 128), lambda i: (0, i, 0)),
            spec,
            spec,
            spec,
            pl.BlockSpec((8, 8), lambda i: (0, 0)),
            pl.BlockSpec((128, 8), lambda i: (0, 0)),
            pl.BlockSpec((8, 128), lambda i: (0, 0)),
            pl.BlockSpec((128, 128), lambda i: (0, 0)),
            pl.BlockSpec((1, 128), lambda i: (0, 0)),
            pl.BlockSpec((1, 128), lambda i: (0, 0)),
        ],
        out_specs=pl.BlockSpec((NUM_GRAPHS, 8), lambda i: (0, 0)),
        out_shape=jax.ShapeDtypeStruct((NUM_GRAPHS, 8), jnp.float32),
    )(acc128, y128, deg128, batch128, wout8, u8, v8, dmask8, bco128, bo128)


def kernel(x, edge_index, batch, emb, W_conv, b_conv, W_out, b_out):
    catp = jnp.concatenate(
        [x[:, 0].astype(jnp.int32), jnp.zeros((NP - N,), jnp.int32)])
    featp = jnp.concatenate([x[:, 1], jnp.zeros((NP - N,), jnp.float32)])
    batchp = jnp.concatenate(
        [batch.astype(jnp.int32), jnp.full((NP - N,), NUM_GRAPHS, jnp.int32)])
    cat128 = jnp.broadcast_to(catp[:, None], (NP, 8)).reshape(RT, 128)
    feat128 = jnp.broadcast_to(featp[:, None], (NP, 8)).reshape(RT, 128)
    batch128 = jnp.broadcast_to(batchp[:, None], (NP, 8)).reshape(RT, 128)
    ei3 = edge_index.astype(jnp.int32).reshape(2, ECH, 128)
    emb48 = jnp.zeros((48, 32), jnp.float32).at[:NUM_CAT].set(emb)
    wc33 = jnp.zeros((33, 8), jnp.float32).at[:, :7].set(W_conv)
    wf128 = jnp.tile(wc33[32], 16).reshape(1, 128)
    bco128 = jnp.tile(jnp.zeros((8,), jnp.float32).at[:7].set(b_conv),
                      16).reshape(1, 128)
    bo128 = jnp.tile(jnp.zeros((8,), jnp.float32).at[:7].set(b_out),
                     16).reshape(1, 128)
    wout8 = jnp.zeros((8, 8), jnp.float32).at[:7, :7].set(W_out)
    i128 = jnp.arange(128)
    i768 = jnp.arange(768)
    pmat = (i128[:, None] == 8 * (i768[None, :] // 48)).astype(jnp.float32)
    umat = (i768[:, None] % 48 == jnp.arange(48)[None, :]).astype(jnp.float32)
    vmat = (jnp.arange(8)[:, None] == i128[None, :] % 8).astype(jnp.float32)
    dmask = (i768[:, None] // 48 == i128[None, :] // 8).astype(jnp.float32)
    u8 = (i128[:, None] % 8 == jnp.arange(8)[None, :]).astype(jnp.float32)
    dmask8 = (i128[:, None] // 8 == i128[None, :] // 8).astype(jnp.float32)
    ones_in = jnp.ones((128,), jnp.float32)
    z1d = jnp.zeros((NS,), jnp.float32)
    z2d = jnp.zeros((896, 8), jnp.float32)

    deg = _run_deg(ei3, ones_in, z1d)
    deg128 = jnp.broadcast_to(deg[:, None], (NP, 8)).reshape(RT, 128)
    y128 = _run_emb(cat128, feat128, deg128, emb48, wc33, wf128, pmat, umat,
                    vmat, dmask)
    acc2 = _run_msg(ei3, y128.reshape(NP, 8), z2d)
    acc128 = acc2.reshape(NSC, RT, 128)

    pooled = _run_pool(acc128, y128, deg128, batch128, wout8, u8, vmat,
                       dmask8, bco128, bo128)
    return pooled[:, :7]


# speedup vs baseline: 101.7139x; 1.0355x over previous
import jax
import jax.numpy as jnp
from jax import lax
from jax.experimental import pallas as pl
from jax.experimental.pallas import tpu as pltpu
from jax.experimental.pallas import tpu_sc as plsc

N = 100000
E = 1600000
NUM_CAT = 43
NUM_GRAPHS = 64

NP = 100352
ECH = E // 128
NSC = 2
NSUB = 16
NW = NSC * NSUB
NS = NP // NSUB
RT = NP * 8 // 128
RB = 128
NBLK = RT // RB


def _deg_body(ei3, ones_in, z1d, deg_out, ebuf_c, onesbuf, deg_sh, ssem):
    sub = lax.axis_index("s")
    pltpu.sync_copy(ones_in, onesbuf)
    pltpu.sync_copy(z1d, deg_sh.at[pl.ds(sub * NS, NS)])
    plsc.subcore_barrier()

    c0 = (ECH * sub) // NSUB
    c1 = (ECH * (sub + 1)) // NSUB
    n8 = (c1 - c0) // 8

    def _deg_group(g, carry):
        base = c0 + 8 * g
        pltpu.sync_copy(ei3.at[1, pl.ds(base, 8)], ebuf_c)
        descs = [
            pltpu.async_copy(onesbuf, deg_sh.at[ebuf_c.at[b]], ssem, add=True)
            for b in range(8)
        ]
        for d in descs:
            d.wait()
        return carry

    lax.fori_loop(0, n8, _deg_group, 0)

    def _deg_tail(i, carry):
        pltpu.sync_copy(ei3.at[1, pl.ds(c0 + 8 * n8 + i, 1)],
                        ebuf_c.at[pl.ds(0, 1)])
        pltpu.async_copy(onesbuf, deg_sh.at[ebuf_c.at[0]], ssem,
                         add=True).wait()
        return carry

    lax.fori_loop(0, (c1 - c0) - 8 * n8, _deg_tail, 0)
    plsc.subcore_barrier()
    core = lax.axis_index("c")

    @pl.when(core == 0)
    def _():
        pltpu.sync_copy(deg_sh.at[pl.ds(sub * NS, NS)],
                        deg_out.at[pl.ds(sub * NS, NS)])


def _run_deg(ei3, ones_in, z1d):
    mesh = plsc.VectorSubcoreMesh(core_axis_name="c", subcore_axis_name="s")
    f = pl.kernel(
        _deg_body,
        out_type=jax.ShapeDtypeStruct((NP,), jnp.float32),
        mesh=mesh,
        scratch_types=[
            pltpu.VMEM((8, 128), jnp.int32),
            pltpu.VMEM((128,), jnp.float32),
            pltpu.VMEM_SHARED((NP,), jnp.float32),
            pltpu.SemaphoreType.DMA,
        ],
        compiler_params=pltpu.CompilerParams(use_tc_tiling_on_sc=False),
    )
    return f(ei3, ones_in, z1d)


def _emb_body(cat_ref, feat_ref, deg_ref, emb_ref, wc_ref, wf_ref,
              p_ref, u_ref, v_ref, dmask_ref, y_ref):
    tbl = jax.lax.dot_general(emb_ref[...], wc_ref[:32, :],
                              (((1,), (0,)), ((), ())),
                              preferred_element_type=jnp.float32)
    tiled = jax.lax.dot_general(
        jax.lax.dot_general(u_ref[...], tbl, (((1,), (0,)), ((), ())),
                            preferred_element_type=jnp.float32),
        v_ref[...], (((1,), (0,)), ((), ())),
        preferred_element_type=jnp.float32)
    tblbd = tiled * dmask_ref[...]
    catf = cat_ref[...].astype(jnp.float32)
    catrep = jax.lax.dot_general(catf, p_ref[...], (((1,), (0,)), ((), ())),
                                 preferred_element_type=jnp.float32)
    mod48 = (lax.broadcasted_iota(jnp.int32, (RB, 768), 1) % 48
             ).astype(jnp.float32)
    ohbig = (catrep == mod48).astype(jnp.float32)
    y0 = jax.lax.dot_general(ohbig, tblbd, (((1,), (0,)), ((), ())),
                             preferred_element_type=jnp.float32)
    dinv = jax.lax.rsqrt(deg_ref[...] + 1.0)
    y_ref[...] = (y0 + feat_ref[...] * wf_ref[...]) * dinv


def _run_emb(cat128, feat128, deg128, emb48, wc33, wf128, pmat, umat, vmat,
             dmask):
    spec = pl.BlockSpec((RB, 128), lambda i: (i, 0))
    return pl.pallas_call(
        _emb_body,
        grid=(NBLK,),
        in_specs=[
            spec,
            spec,
            spec,
            pl.BlockSpec((48, 32), lambda i: (0, 0)),
            pl.BlockSpec((33, 8), lambda i: (0, 0)),
            pl.BlockSpec((1, 128), lambda i: (0, 0)),
            pl.BlockSpec((128, 768), lambda i: (0, 0)),
            pl.BlockSpec((768, 48), lambda i: (0, 0)),
            pl.BlockSpec((8, 128), lambda i: (0, 0)),
            pl.BlockSpec((768, 128), lambda i: (0, 0)),
        ],
        out_specs=spec,
        out_shape=jax.ShapeDtypeStruct((RT, 128), jnp.float32),
    )(cat128, feat128, deg128, emb48, wc33, wf128, pmat, umat, vmat, dmask)


def _msg_body(ei3, y_hbm, z2d, acc_out,
              ebuf_r, ebuf_c, gbuf, y_sh, acc_sh, gsem, ssem):
    core = lax.axis_index("c")
    sub = lax.axis_index("s")
    wid = core * NSUB + sub

    pltpu.sync_copy(y_hbm.at[pl.ds(sub * NS, NS)],
                    y_sh.at[pl.ds(sub * NS, NS)])
    for k in range(7):
        pltpu.sync_copy(z2d, acc_sh.at[pl.ds(sub * NS + 896 * k, 896)])
    plsc.subcore_barrier()

    c0 = (ECH * wid) // NW
    c1 = (ECH * (wid + 1)) // NW
    n8 = (c1 - c0) // 8

    def _load_edges(g, d):
        pltpu.sync_copy(ei3.at[0, pl.ds(c0 + 8 * g, 8)], ebuf_r.at[d])
        pltpu.sync_copy(ei3.at[1, pl.ds(c0 + 8 * g, 8)], ebuf_c.at[d])

    def _fire_gathers(d):
        return [
            pltpu.async_copy(y_sh.at[ebuf_r.at[d, b]],
                             gbuf.at[d, pl.ds(128 * b, 128)], gsem)
            for b in range(8)
        ]

    def _fire_scatters(d):
        return [
            pltpu.async_copy(gbuf.at[d, pl.ds(128 * b, 128)],
                             acc_sh.at[ebuf_c.at[d, b]], ssem, add=True)
            for b in range(8)
        ]

    @pl.when(n8 > 0)
    def _():
        _load_edges(0, 0)
        for d in _fire_gathers(0):
            d.wait()

        def _msg_group(g, carry):
            cur = lax.rem(g, 2)
            nxt = 1 - cur
            has_next = g + 1 < n8

            @pl.when(has_next)
            def _():
                _load_edges(g + 1, nxt)

            sds = _fire_scatters(cur)

            @pl.when(has_next)
            def _():
                for d in _fire_gathers(nxt):
                    d.wait()

            for d in sds:
                d.wait()
            return carry

        lax.fori_loop(0, n8, _msg_group, 0)

    def _msg_tail(i, carry):
        pltpu.sync_copy(ei3.at[0, pl.ds(c0 + 8 * n8 + i, 1)],
                        ebuf_r.at[0, pl.ds(0, 1)])
        pltpu.sync_copy(ei3.at[1, pl.ds(c0 + 8 * n8 + i, 1)],
                        ebuf_c.at[0, pl.ds(0, 1)])
        pltpu.async_copy(y_sh.at[ebuf_r.at[0, 0]], gbuf.at[0, pl.ds(0, 128)],
                         gsem).wait()
        pltpu.async_copy(gbuf.at[0, pl.ds(0, 128)], acc_sh.at[ebuf_c.at[0, 0]],
                         ssem, add=True).wait()
        return carry

    lax.fori_loop(0, (c1 - c0) - 8 * n8, _msg_tail, 0)
    plsc.subcore_barrier()
    pltpu.sync_copy(acc_sh.at[pl.ds(sub * NS, NS)],
                    acc_out.at[core, pl.ds(sub * NS, NS)])


def _run_msg(ei3, y, z2d):
    mesh = plsc.VectorSubcoreMesh(core_axis_name="c", subcore_axis_name="s")
    f = pl.kernel(
        _msg_body,
        out_type=jax.ShapeDtypeStruct((NSC, NP, 8), jnp.float32),
        mesh=mesh,
        scratch_types=[
            pltpu.VMEM((2, 8, 128), jnp.int32),
            pltpu.VMEM((2, 8, 128), jnp.int32),
            pltpu.VMEM((2, 1024, 8), jnp.float32),
            pltpu.VMEM_SHARED((NP, 8), jnp.float32),
            pltpu.VMEM_SHARED((NP, 8), jnp.float32),
            pltpu.SemaphoreType.DMA,
            pltpu.SemaphoreType.DMA,
        ],
        compiler_params=pltpu.CompilerParams(use_tc_tiling_on_sc=False),
    )
    return f(ei3, y, z2d)


def _pool_body(acc_ref, y_ref, deg_ref, batch_ref, wout_ref, u8_ref, v_ref,
               dmask8_ref, bco_ref, bo_ref, out_ref):
    i = pl.program_id(0)
    dinv = jax.lax.rsqrt(deg_ref[...] + 1.0)
    conv = dinv * (acc_ref[0] + acc_ref[1] + y_ref[...]) + bco_ref[...]
    wbd = jax.lax.dot_general(
        jax.lax.dot_general(u8_ref[...], wout_ref[...],
                            (((1,), (0,)), ((), ())),
                            preferred_element_type=jnp.float32),
        v_ref[...], (((1,), (0,)), ((), ())),
        preferred_element_type=jnp.float32) * dmask8_ref[...]
    z = jax.lax.dot_general(conv, wbd, (((1,), (0,)), ((), ())),
                            preferred_element_type=jnp.float32)
    z = jnp.maximum(z + bo_ref[...], 0.0)
    zext = z + (lax.broadcasted_iota(jnp.int32, (RB, 128), 1) % 8 == 7
                ).astype(jnp.float32)
    iota64 = lax.broadcasted_iota(jnp.int32, (1, NUM_GRAPHS), 1)
    p = jnp.zeros((NUM_GRAPHS, 8), jnp.float32)
    for s in range(16):
        z_s = zext[:, 8 * s:8 * s + 8]
        b_s = batch_ref[:, 8 * s:8 * s + 1]
        oh_s = (b_s == iota64).astype(jnp.float32)
        p = p + jax.lax.dot_general(oh_s, z_s, (((0,), (0,)), ((), ())),
                                    preferred_element_type=jnp.float32)

    @pl.when(i == 0)
    def _():
        out_ref[...] = p

    @pl.when(i > 0)
    def _():
        out_ref[...] = out_ref[...] + p

    @pl.when(i == NBLK - 1)
    def _():
        f = out_ref[...]
        out_ref[...] = f / jnp.maximum(f[:, 7:8], 1.0)


def _run_pool(acc128, y128, deg128, batch128, wout8, u8, v8, dmask8, bco128,
              bo128):
    spec = pl.BlockSpec((RB, 128), lambda i: (i, 0))
    return pl.pallas_call(
        _pool_body,
        grid=(NBLK,),
        in_specs=[
            pl.BlockSpec((NSC, RB, 128), lambda i: (0, i, 0)),
            spec,
            spec,
            spec,
            pl.BlockSpec((8, 8), lambda i: (0, 0)),
            pl.BlockSpec((128, 8), lambda i: (0, 0)),
            pl.BlockSpec((8, 128), lambda i: (0, 0)),
            pl.BlockSpec((128, 128), lambda i: (0, 0)),
            pl.BlockSpec((1, 128), lambda i: (0, 0)),
            pl.BlockSpec((1, 128), lambda i: (0, 0)),
        ],
        out_specs=pl.BlockSpec((NUM_GRAPHS, 8), lambda i: (0, 0)),
        out_shape=jax.ShapeDtypeStruct((NUM_GRAPHS, 8), jnp.float32),
    )(acc128, y128, deg128, batch128, wout8, u8, v8, dmask8, bco128, bo128)


def kernel(x, edge_index, batch, emb, W_conv, b_conv, W_out, b_out):
    catp = jnp.concatenate(
        [x[:, 0].astype(jnp.int32), jnp.zeros((NP - N,), jnp.int32)])
    featp = jnp.concatenate([x[:, 1], jnp.zeros((NP - N,), jnp.float32)])
    batchp = jnp.concatenate(
        [batch.astype(jnp.int32), jnp.full((NP - N,), NUM_GRAPHS, jnp.int32)])
    cat128 = jnp.broadcast_to(catp[:, None], (NP, 8)).reshape(RT, 128)
    feat128 = jnp.broadcast_to(featp[:, None], (NP, 8)).reshape(RT, 128)
    batch128 = jnp.broadcast_to(batchp[:, None], (NP, 8)).reshape(RT, 128)
    ei3 = edge_index.astype(jnp.int32).reshape(2, ECH, 128)
    emb48 = jnp.zeros((48, 32), jnp.float32).at[:NUM_CAT].set(emb)
    wc33 = jnp.zeros((33, 8), jnp.float32).at[:, :7].set(W_conv)
    wf128 = jnp.tile(wc33[32], 16).reshape(1, 128)
    bco128 = jnp.tile(jnp.zeros((8,), jnp.float32).at[:7].set(b_conv),
                      16).reshape(1, 128)
    bo128 = jnp.tile(jnp.zeros((8,), jnp.float32).at[:7].set(b_out),
                     16).reshape(1, 128)
    wout8 = jnp.zeros((8, 8), jnp.float32).at[:7, :7].set(W_out)
    i128 = jnp.arange(128)
    i768 = jnp.arange(768)
    pmat = (i128[:, None] == 8 * (i768[None, :] // 48)).astype(jnp.float32)
    umat = (i768[:, None] % 48 == jnp.arange(48)[None, :]).astype(jnp.float32)
    vmat = (jnp.arange(8)[:, None] == i128[None, :] % 8).astype(jnp.float32)
    dmask = (i768[:, None] // 48 == i128[None, :] // 8).astype(jnp.float32)
    u8 = (i128[:, None] % 8 == jnp.arange(8)[None, :]).astype(jnp.float32)
    dmask8 = (i128[:, None] // 8 == i128[None, :] // 8).astype(jnp.float32)
    ones_in = jnp.ones((128,), jnp.float32)
    z1d = jnp.zeros((NS,), jnp.float32)
    z2d = jnp.zeros((896, 8), jnp.float32)

    deg = _run_deg(ei3, ones_in, z1d)
    deg128 = jnp.broadcast_to(deg[:, None], (NP, 8)).reshape(RT, 128)
    y128 = _run_emb(cat128, feat128, deg128, emb48, wc33, wf128, pmat, umat,
                    vmat, dmask)
    acc2 = _run_msg(ei3, y128.reshape(NP, 8), z2d)
    acc128 = acc2.reshape(NSC, RT, 128)

    pooled = _run_pool(acc128, y128, deg128, batch128, wout8, u8, vmat,
                       dmask8, bco128, bo128)
    return pooled[:, :7]


# speedup vs baseline: 110.9069x; 1.0904x over previous
import jax
import jax.numpy as jnp
from jax import lax
from jax.experimental import pallas as pl
from jax.experimental.pallas import tpu as pltpu
from jax.experimental.pallas import tpu_sc as plsc

N = 100000
E = 1600000
NUM_CAT = 43
NUM_GRAPHS = 64

NP = 100352
ECH = E // 128
NSC = 2
NSUB = 16
NW = NSC * NSUB
NS = NP // NSUB
RT = NP * 8 // 128
RB = 128
NBLK = RT // RB


def _deg_body(ei3, ones_in, z1d, deg_out, ebuf_c, onesbuf, deg_sh, ssem):
    sub = lax.axis_index("s")
    pltpu.sync_copy(ones_in, onesbuf)
    pltpu.sync_copy(z1d, deg_sh.at[pl.ds(sub * NS, NS)])
    plsc.subcore_barrier()

    c0 = (ECH * sub) // NSUB
    c1 = (ECH * (sub + 1)) // NSUB
    n8 = (c1 - c0) // 8

    @pl.when(n8 > 0)
    def _():
        pltpu.sync_copy(ei3.at[1, pl.ds(c0, 8)], ebuf_c.at[0])

        def _deg_group(g, carry):
            cur = lax.rem(g, 2)
            descs = [
                pltpu.async_copy(onesbuf, deg_sh.at[ebuf_c.at[cur, b]],
                                 ssem, add=True)
                for b in range(8)
            ]

            @pl.when(g + 1 < n8)
            def _():
                pltpu.sync_copy(ei3.at[1, pl.ds(c0 + 8 * (g + 1), 8)],
                                ebuf_c.at[1 - cur])

            for d in descs:
                d.wait()
            return carry

        lax.fori_loop(0, n8, _deg_group, 0)

    def _deg_tail(i, carry):
        pltpu.sync_copy(ei3.at[1, pl.ds(c0 + 8 * n8 + i, 1)],
                        ebuf_c.at[0, pl.ds(0, 1)])
        pltpu.async_copy(onesbuf, deg_sh.at[ebuf_c.at[0, 0]], ssem,
                         add=True).wait()
        return carry

    lax.fori_loop(0, (c1 - c0) - 8 * n8, _deg_tail, 0)
    plsc.subcore_barrier()
    core = lax.axis_index("c")

    @pl.when(core == 0)
    def _():
        pltpu.sync_copy(deg_sh.at[pl.ds(sub * NS, NS)],
                        deg_out.at[pl.ds(sub * NS, NS)])


def _run_deg(ei3, ones_in, z1d):
    mesh = plsc.VectorSubcoreMesh(core_axis_name="c", subcore_axis_name="s")
    f = pl.kernel(
        _deg_body,
        out_type=jax.ShapeDtypeStruct((NP,), jnp.float32),
        mesh=mesh,
        scratch_types=[
            pltpu.VMEM((2, 8, 128), jnp.int32),
            pltpu.VMEM((128,), jnp.float32),
            pltpu.VMEM_SHARED((NP,), jnp.float32),
            pltpu.SemaphoreType.DMA,
        ],
        compiler_params=pltpu.CompilerParams(use_tc_tiling_on_sc=False),
    )
    return f(ei3, ones_in, z1d)


def _emb_body(cat_ref, feat_ref, deg_ref, emb_ref, wc_ref, wf_ref,
              p_ref, u_ref, v_ref, dmask_ref, y_ref):
    tbl = jax.lax.dot_general(emb_ref[...], wc_ref[:32, :],
                              (((1,), (0,)), ((), ())),
                              preferred_element_type=jnp.float32)
    tiled = jax.lax.dot_general(
        jax.lax.dot_general(u_ref[...], tbl, (((1,), (0,)), ((), ())),
                            preferred_element_type=jnp.float32),
        v_ref[...], (((1,), (0,)), ((), ())),
        preferred_element_type=jnp.float32)
    tblbd = tiled * dmask_ref[...]
    catf = cat_ref[...].astype(jnp.float32)
    catrep = jax.lax.dot_general(catf, p_ref[...], (((1,), (0,)), ((), ())),
                                 preferred_element_type=jnp.float32)
    mod48 = (lax.broadcasted_iota(jnp.int32, (RB, 768), 1) % 48
             ).astype(jnp.float32)
    ohbig = (catrep == mod48).astype(jnp.float32)
    y0 = jax.lax.dot_general(ohbig, tblbd, (((1,), (0,)), ((), ())),
                             preferred_element_type=jnp.float32)
    dinv = jax.lax.rsqrt(deg_ref[...] + 1.0)
    y_ref[...] = (y0 + feat_ref[...] * wf_ref[...]) * dinv


def _run_emb(cat128, feat128, deg128, emb48, wc33, wf128, pmat, umat, vmat,
             dmask):
    spec = pl.BlockSpec((RB, 128), lambda i: (i, 0))
    return pl.pallas_call(
        _emb_body,
        grid=(NBLK,),
        in_specs=[
            spec,
            spec,
            spec,
            pl.BlockSpec((48, 32), lambda i: (0, 0)),
            pl.BlockSpec((33, 8), lambda i: (0, 0)),
            pl.BlockSpec((1, 128), lambda i: (0, 0)),
            pl.BlockSpec((128, 768), lambda i: (0, 0)),
            pl.BlockSpec((768, 48), lambda i: (0, 0)),
            pl.BlockSpec((8, 128), lambda i: (0, 0)),
            pl.BlockSpec((768, 128), lambda i: (0, 0)),
        ],
        out_specs=spec,
        out_shape=jax.ShapeDtypeStruct((RT, 128), jnp.float32),
    )(cat128, feat128, deg128, emb48, wc33, wf128, pmat, umat, vmat, dmask)


def _msg_body(ei3, y_hbm, z2d, acc_out,
              ebuf_r, ebuf_c, gbuf, y_sh, acc_sh, gsem, ssem):
    core = lax.axis_index("c")
    sub = lax.axis_index("s")
    wid = core * NSUB + sub

    pltpu.sync_copy(y_hbm.at[pl.ds(sub * NS, NS)],
                    y_sh.at[pl.ds(sub * NS, NS)])
    for k in range(7):
        pltpu.sync_copy(z2d, acc_sh.at[pl.ds(sub * NS + 896 * k, 896)])
    plsc.subcore_barrier()

    c0 = (ECH * wid) // NW
    c1 = (ECH * (wid + 1)) // NW
    n8 = (c1 - c0) // 8

    def _load_edges(g, d):
        pltpu.sync_copy(ei3.at[0, pl.ds(c0 + 8 * g, 8)], ebuf_r.at[d])
        pltpu.sync_copy(ei3.at[1, pl.ds(c0 + 8 * g, 8)], ebuf_c.at[d])

    def _fire_gathers(d):
        return [
            pltpu.async_copy(y_sh.at[ebuf_r.at[d, b]],
                             gbuf.at[d, pl.ds(128 * b, 128)], gsem)
            for b in range(8)
        ]

    def _fire_scatters(d):
        return [
            pltpu.async_copy(gbuf.at[d, pl.ds(128 * b, 128)],
                             acc_sh.at[ebuf_c.at[d, b]], ssem, add=True)
            for b in range(8)
        ]

    @pl.when(n8 > 0)
    def _():
        _load_edges(0, 0)
        for d in _fire_gathers(0):
            d.wait()

        def _msg_group(g, carry):
            cur = lax.rem(g, 2)
            nxt = 1 - cur
            has_next = g + 1 < n8

            @pl.when(has_next)
            def _():
                _load_edges(g + 1, nxt)

            sds = _fire_scatters(cur)

            @pl.when(has_next)
            def _():
                for d in _fire_gathers(nxt):
                    d.wait()

            for d in sds:
                d.wait()
            return carry

        lax.fori_loop(0, n8, _msg_group, 0)

    def _msg_tail(i, carry):
        pltpu.sync_copy(ei3.at[0, pl.ds(c0 + 8 * n8 + i, 1)],
                        ebuf_r.at[0, pl.ds(0, 1)])
        pltpu.sync_copy(ei3.at[1, pl.ds(c0 + 8 * n8 + i, 1)],
                        ebuf_c.at[0, pl.ds(0, 1)])
        pltpu.async_copy(y_sh.at[ebuf_r.at[0, 0]], gbuf.at[0, pl.ds(0, 128)],
                         gsem).wait()
        pltpu.async_copy(gbuf.at[0, pl.ds(0, 128)], acc_sh.at[ebuf_c.at[0, 0]],
                         ssem, add=True).wait()
        return carry

    lax.fori_loop(0, (c1 - c0) - 8 * n8, _msg_tail, 0)
    plsc.subcore_barrier()
    pltpu.sync_copy(acc_sh.at[pl.ds(sub * NS, NS)],
                    acc_out.at[core, pl.ds(sub * NS, NS)])


def _run_msg(ei3, y, z2d):
    mesh = plsc.VectorSubcoreMesh(core_axis_name="c", subcore_axis_name="s")
    f = pl.kernel(
        _msg_body,
        out_type=jax.ShapeDtypeStruct((NSC, NP, 8), jnp.float32),
        mesh=mesh,
        scratch_types=[
            pltpu.VMEM((2, 8, 128), jnp.int32),
            pltpu.VMEM((2, 8, 128), jnp.int32),
            pltpu.VMEM((2, 1024, 8), jnp.float32),
            pltpu.VMEM_SHARED((NP, 8), jnp.float32),
            pltpu.VMEM_SHARED((NP, 8), jnp.float32),
            pltpu.SemaphoreType.DMA,
            pltpu.SemaphoreType.DMA,
        ],
        compiler_params=pltpu.CompilerParams(use_tc_tiling_on_sc=False),
    )
    return f(ei3, y, z2d)


def _pool_body(acc_ref, y_ref, deg_ref, batch_ref, wout_ref, u8_ref, v_ref,
               dmask8_ref, bco_ref, bo_ref, out_ref):
    i = pl.program_id(0)
    dinv = jax.lax.rsqrt(deg_ref[...] + 1.0)
    conv = dinv * (acc_ref[0] + acc_ref[1] + y_ref[...]) + bco_ref[...]
    wbd = jax.lax.dot_general(
        jax.lax.dot_general(u8_ref[...], wout_ref[...],
                            (((1,), (0,)), ((), ())),
                            preferred_element_type=jnp.float32),
        v_ref[...], (((1,), (0,)), ((), ())),
        preferred_element_type=jnp.float32) * dmask8_ref[...]
    z = jax.lax.dot_general(conv, wbd, (((1,), (0,)), ((), ())),
                            preferred_element_type=jnp.float32)
    z = jnp.maximum(z + bo_ref[...], 0.0)
    zext = z + (lax.broadcasted_iota(jnp.int32, (RB, 128), 1) % 8 == 7
                ).astype(jnp.float32)
    iota64 = lax.broadcasted_iota(jnp.int32, (1, NUM_GRAPHS), 1)
    p = jnp.zeros((NUM_GRAPHS, 8), jnp.float32)
    for s in range(16):
        z_s = zext[:, 8 * s:8 * s + 8]
        b_s = batch_ref[:, 8 * s:8 * s + 1]
        oh_s = (b_s == iota64).astype(jnp.float32)
        p = p + jax.lax.dot_general(oh_s, z_s, (((0,), (0,)), ((), ())),
                                    preferred_element_type=jnp.float32)

    @pl.when(i == 0)
    def _():
        out_ref[...] = p

    @pl.when(i > 0)
    def _():
        out_ref[...] = out_ref[...] + p

    @pl.when(i == NBLK - 1)
    def _():
        f = out_ref[...]
        out_ref[...] = f / jnp.maximum(f[:, 7:8], 1.0)


def _run_pool(acc128, y128, deg128, batch128, wout8, u8, v8, dmask8, bco128,
              bo128):
    spec = pl.BlockSpec((RB, 128), lambda i: (i, 0))
    return pl.pallas_call(
        _pool_body,
        grid=(NBLK,),
        in_specs=[
            pl.BlockSpec((NSC, RB, 128), lambda i: (0, i, 0)),
            spec,
            spec,
            spec,
            pl.BlockSpec((8, 8), lambda i: (0, 0)),
            pl.BlockSpec((128, 8), lambda i: (0, 0)),
            pl.BlockSpec((8, 128), lambda i: (0, 0)),
            pl.BlockSpec((128, 128), lambda i: (0, 0)),
            pl.BlockSpec((1, 128), lambda i: (0, 0)),
            pl.BlockSpec((1, 128), lambda i: (0, 0)),
        ],
        out_specs=pl.BlockSpec((NUM_GRAPHS, 8), lambda i: (0, 0)),
        out_shape=jax.ShapeDtypeStruct((NUM_GRAPHS, 8), jnp.float32),
    )(acc128, y128, deg128, batch128, wout8, u8, v8, dmask8, bco128, bo128)


def kernel(x, edge_index, batch, emb, W_conv, b_conv, W_out, b_out):
    catp = jnp.concatenate(
        [x[:, 0].astype(jnp.int32), jnp.zeros((NP - N,), jnp.int32)])
    featp = jnp.concatenate([x[:, 1], jnp.zeros((NP - N,), jnp.float32)])
    batchp = jnp.concatenate(
        [batch.astype(jnp.int32), jnp.full((NP - N,), NUM_GRAPHS, jnp.int32)])
    cat128 = jnp.broadcast_to(catp[:, None], (NP, 8)).reshape(RT, 128)
    feat128 = jnp.broadcast_to(featp[:, None], (NP, 8)).reshape(RT, 128)
    batch128 = jnp.broadcast_to(batchp[:, None], (NP, 8)).reshape(RT, 128)
    ei3 = edge_index.astype(jnp.int32).reshape(2, ECH, 128)
    emb48 = jnp.zeros((48, 32), jnp.float32).at[:NUM_CAT].set(emb)
    wc33 = jnp.zeros((33, 8), jnp.float32).at[:, :7].set(W_conv)
    wf128 = jnp.tile(wc33[32], 16).reshape(1, 128)
    bco128 = jnp.tile(jnp.zeros((8,), jnp.float32).at[:7].set(b_conv),
                      16).reshape(1, 128)
    bo128 = jnp.tile(jnp.zeros((8,), jnp.float32).at[:7].set(b_out),
                     16).reshape(1, 128)
    wout8 = jnp.zeros((8, 8), jnp.float32).at[:7, :7].set(W_out)
    i128 = jnp.arange(128)
    i768 = jnp.arange(768)
    pmat = (i128[:, None] == 8 * (i768[None, :] // 48)).astype(jnp.float32)
    umat = (i768[:, None] % 48 == jnp.arange(48)[None, :]).astype(jnp.float32)
    vmat = (jnp.arange(8)[:, None] == i128[None, :] % 8).astype(jnp.float32)
    dmask = (i768[:, None] // 48 == i128[None, :] // 8).astype(jnp.float32)
    u8 = (i128[:, None] % 8 == jnp.arange(8)[None, :]).astype(jnp.float32)
    dmask8 = (i128[:, None] // 8 == i128[None, :] // 8).astype(jnp.float32)
    ones_in = jnp.ones((128,), jnp.float32)
    z1d = jnp.zeros((NS,), jnp.float32)
    z2d = jnp.zeros((896, 8), jnp.float32)

    deg = _run_deg(ei3, ones_in, z1d)
    deg128 = jnp.broadcast_to(deg[:, None], (NP, 8)).reshape(RT, 128)
    y128 = _run_emb(cat128, feat128, deg128, emb48, wc33, wf128, pmat, umat,
                    vmat, dmask)
    acc2 = _run_msg(ei3, y128.reshape(NP, 8), z2d)
    acc128 = acc2.reshape(NSC, RT, 128)

    pooled = _run_pool(acc128, y128, deg128, batch128, wout8, u8, vmat,
                       dmask8, bco128, bo128)
    return pooled[:, :7]
